# Initial kernel scaffold; baseline (speedup 1.0000x reference)
import jax
import jax.numpy as jnp
from jax.experimental import pallas as pl


def kernel(ST_fit, ST_supervision, ST_edge, SC_fit, SC_supervision, SC_label, SC_genegraph, params):
    raise NotImplementedError("write your pallas kernel here")


# speedup vs baseline: 1.0169x; 1.0169x over previous
import functools
import jax
import jax.numpy as jnp
from jax.experimental import pallas as pl
from jax.experimental.pallas import tpu as pltpu

DI = 2000
DH = 512
DL = 32
NC = 20
KK = 50
NS = 2048
NE = 12288
DS = 1000
RB = 256
NBLK = NS // RB


def _f32(*shape):
    return jax.ShapeDtypeStruct(shape, jnp.float32)


def _enc_body(st_ref, sc_ref, wg_ref, wr_ref, asrc_ref, adst_ref,
              we1_ref, be1_ref, we2_ref, be2_ref,
              h_ref, r_ref, es_ref, ed_ref, scl_ref):
    x = st_ref[...]
    h = jnp.dot(x, wg_ref[...], preferred_element_type=jnp.float32)
    h_ref[...] = h
    r_ref[...] = jnp.dot(x, wr_ref[...], preferred_element_type=jnp.float32)
    es_ref[...] = jnp.dot(h, asrc_ref[...], preferred_element_type=jnp.float32)
    ed_ref[...] = jnp.dot(h, adst_ref[...], preferred_element_type=jnp.float32)
    xc = sc_ref[...]
    hc = jnp.maximum(
        jnp.dot(xc, we1_ref[...], preferred_element_type=jnp.float32)
        + be1_ref[...], 0.0)
    scl_ref[...] = (jnp.dot(hc, we2_ref[...], preferred_element_type=jnp.float32)
                    + be2_ref[...])


def _encoders(ST_fit, SC_fit, p):
    blk = lambda i: (i, 0)
    full = lambda i: (0, 0)
    return pl.pallas_call(
        _enc_body,
        grid=(NBLK,),
        in_specs=[
            pl.BlockSpec((RB, DI), blk),
            pl.BlockSpec((RB, DI), blk),
            pl.BlockSpec((DI, DH), full),
            pl.BlockSpec((DI, DH), full),
            pl.BlockSpec((DH, 1), full),
            pl.BlockSpec((DH, 1), full),
            pl.BlockSpec((DI, DH), full),
            pl.BlockSpec((1, DH), full),
            pl.BlockSpec((DH, DL), full),
            pl.BlockSpec((1, DL), full),
        ],
        out_specs=[
            pl.BlockSpec((RB, DH), blk),
            pl.BlockSpec((RB, DH), blk),
            pl.BlockSpec((RB, 1), blk),
            pl.BlockSpec((RB, 1), blk),
            pl.BlockSpec((RB, DL), blk),
        ],
        out_shape=[_f32(NS, DH), _f32(NS, DH), _f32(NS, 1), _f32(NS, 1),
                   _f32(NS, DL)],
        compiler_params=pltpu.CompilerParams(vmem_limit_bytes=100 << 20),
    )(ST_fit, SC_fit, p['W_gat'], p['W_res'],
      p['a_src'].reshape(DH, 1), p['a_dst'].reshape(DH, 1),
      p['We1'], p['be1'].reshape(1, DH), p['We2'], p['be2'].reshape(1, DL))


def _lat_body(agg_ref, r_ref, wl_ref, bl_ref, scl_ref, lab_ref,
              wc1_ref, bc1_ref, wc2_ref, bc2_ref,
              stl_ref, mmd_ref, clf_ref):
    pre = agg_ref[...] + r_ref[...]
    hout = jnp.where(pre > 0, pre, jnp.exp(jnp.minimum(pre, 0.0)) - 1.0)
    stl = (jnp.dot(hout, wl_ref[...], preferred_element_type=jnp.float32)
           + bl_ref[...])
    stl_ref[...] = stl

    scl = scl_ref[...]
    t1 = jnp.maximum(
        jnp.dot(scl, wc1_ref[...], preferred_element_type=jnp.float32)
        + bc1_ref[...], 0.0)
    logits = (jnp.dot(t1, wc2_ref[...], preferred_element_type=jnp.float32)
              + bc2_ref[...])
    mx = jnp.max(logits, axis=1, keepdims=True)
    sh = logits - mx
    ls = sh - jnp.log(jnp.sum(jnp.exp(sh), axis=1, keepdims=True))
    cols = jax.lax.broadcasted_iota(jnp.int32, (NS, NC), 1)
    sel = jnp.where(cols == lab_ref[...], ls, 0.0)
    clf_ref[...] = (-jnp.sum(sel) / NS).reshape(1, 1)

    q = stl
    k = scl
    qs = jnp.sum(q * q, axis=1, keepdims=True)
    ks = jnp.sum(k * k, axis=1, keepdims=True)

    def dblock(a, b, sb):
        sa = jnp.sum(a * a, axis=1, keepdims=True)
        ab = jax.lax.dot_general(
            a, b, (((1,), (1,)), ((), ())), preferred_element_type=jnp.float32)
        return jnp.maximum(sa + sb.reshape(1, NS) - 2.0 * ab, 0.0)

    def bw_step(i, acc):
        qb = stl_ref[pl.ds(i * RB, RB), :]
        d = dblock(qb, k, ks)
        return acc + jnp.sum(d)

    bwsum = jax.lax.fori_loop(0, NBLK, bw_step, 0.0)
    bw = bwsum / (NS * NS) + 1e-12

    def exp_step(i, accs):
        axx, ayy, axy = accs
        qb = stl_ref[pl.ds(i * RB, RB), :]
        kb = scl_ref[pl.ds(i * RB, RB), :]
        dxx = dblock(qb, q, qs)
        dyy = dblock(kb, k, ks)
        dxy = dblock(qb, k, ks)
        axx = axx + jnp.sum(jnp.exp(-dxx / bw))
        ayy = ayy + jnp.sum(jnp.exp(-dyy / bw))
        axy = axy + jnp.sum(jnp.exp(-dxy / bw))
        return axx, ayy, axy

    sxx, syy, sxy = jax.lax.fori_loop(0, NBLK, exp_step, (0.0, 0.0, 0.0))
    mmd_ref[...] = ((sxx + syy - 2.0 * sxy) / (NS * NS)).reshape(1, 1)


def _latent_mmd_clf(agg, r, scl, lab, p):
    full = lambda: (0, 0)
    spec = lambda s: pl.BlockSpec(s, lambda: (0, 0))
    return pl.pallas_call(
        _lat_body,
        in_specs=[spec((NS, DH)), spec((NS, DH)), spec((DH, DL)),
                  spec((1, DL)), spec((NS, DL)), spec((NS, 1)),
                  spec((DL, DH)), spec((1, DH)), spec((DH, NC)),
                  spec((1, NC))],
        out_specs=[spec((NS, DL)), spec((1, 1)), spec((1, 1))],
        out_shape=[_f32(NS, DL), _f32(1, 1), _f32(1, 1)],
        compiler_params=pltpu.CompilerParams(vmem_limit_bytes=100 << 20),
    )(agg, r, p['W_lat'], p['b_lat'].reshape(1, DL), scl,
      lab.reshape(NS, 1), p['Wc1'], p['bc1'].reshape(1, DH),
      p['Wc2'], p['bc2'].reshape(1, NC))


def _dec_body(stl_ref, scl_ref, st_ref, sc_ref, sup_ref, imp_ref,
              w1st_ref, b1st_ref, w2st_ref, b2st_ref,
              w1sc_ref, b1sc_ref, w2sc_ref, b2sc_ref,
              rec_ref, cos_ref, imp_out_ref):
    i = pl.program_id(0)

    def dec(z, w1, b1, w2, b2):
        d1 = jnp.maximum(
            jnp.dot(z, w1, preferred_element_type=jnp.float32) + b1, 0.0)
        return jnp.dot(d1, w2, preferred_element_type=jnp.float32) + b2

    z = stl_ref[...]
    z2 = scl_ref[...]
    xst = st_ref[...]
    xsc = sc_ref[...]
    w1st = w1st_ref[...]; b1st = b1st_ref[...]
    w2st = w2st_ref[...]; b2st = b2st_ref[...]
    w1sc = w1sc_ref[...]; b1sc = b1sc_ref[...]
    w2sc = w2sc_ref[...]; b2sc = b2sc_ref[...]

    rec_st = dec(z, w1st, b1st, w2st, b2st)
    rec_sc = dec(z2, w1sc, b1sc, w2sc, b2sc)
    d1 = xst - rec_st
    d2 = xsc - rec_sc
    rsum = jnp.sum(d1 * d1) + jnp.sum(d2 * d2)

    def cosrows(a, b):
        na = jnp.sqrt(jnp.sum(a * a, axis=1, keepdims=True)) + 1e-12
        nb = jnp.sqrt(jnp.sum(b * b, axis=1, keepdims=True)) + 1e-12
        c = jnp.sum((a / na) * (b / nb), axis=1)
        return jnp.sum(1.0 - c)

    cross_sc = dec(z, w1sc, b1sc, w2sc, b2sc)
    cross_st = dec(z2, w1st, b1st, w2st, b2st)
    csum = cosrows(cross_sc, xst) + cosrows(cross_st, xsc)

    P = imp_ref[...]
    T = sup_ref[...]
    Pc = P - jnp.mean(P, axis=1, keepdims=True)
    Tc = T - jnp.mean(T, axis=1, keepdims=True)
    num = jnp.sum(Pc * Tc, axis=1)
    den = (jnp.sqrt(jnp.sum(Pc * Pc, axis=1)) *
           jnp.sqrt(jnp.sum(Tc * Tc, axis=1)) + 1e-12)
    isum = jnp.sum(1.0 - num / den)

    @pl.when(i == 0)
    def _():
        rec_ref[...] = jnp.zeros((1, 1), jnp.float32)
        cos_ref[...] = jnp.zeros((1, 1), jnp.float32)
        imp_out_ref[...] = jnp.zeros((1, 1), jnp.float32)

    rec_ref[...] += rsum.reshape(1, 1)
    cos_ref[...] += csum.reshape(1, 1)
    imp_out_ref[...] += isum.reshape(1, 1)


def _decoders(stl, scl, ST_fit, SC_fit, sup, imp, p):
    blk = lambda i: (i, 0)
    full = lambda i: (0, 0)
    return pl.pallas_call(
        _dec_body,
        grid=(NBLK,),
        in_specs=[
            pl.BlockSpec((RB, DL), blk),
            pl.BlockSpec((RB, DL), blk),
            pl.BlockSpec((RB, DI), blk),
            pl.BlockSpec((RB, DI), blk),
            pl.BlockSpec((RB, DS), blk),
            pl.BlockSpec((RB, DS), blk),
            pl.BlockSpec((DL, DH), full),
            pl.BlockSpec((1, DH), full),
            pl.BlockSpec((DH, DI), full),
            pl.BlockSpec((1, DI), full),
            pl.BlockSpec((DL, DH), full),
            pl.BlockSpec((1, DH), full),
            pl.BlockSpec((DH, DI), full),
            pl.BlockSpec((1, DI), full),
        ],
        out_specs=[pl.BlockSpec((1, 1), full)] * 3,
        out_shape=[_f32(1, 1)] * 3,
        compiler_params=pltpu.CompilerParams(vmem_limit_bytes=100 << 20),
    )(stl, scl, ST_fit, SC_fit, sup, imp,
      p['Wd1_st'], p['bd1_st'].reshape(1, DH), p['Wd2_st'],
      p['bd2_st'].reshape(1, DI),
      p['Wd1_sc'], p['bd1_sc'].reshape(1, DH), p['Wd2_sc'],
      p['bd2_sc'].reshape(1, DI))


def _gg_body(st_ref, imp_ref, gg_ref, out_ref):
    P = imp_ref[...]
    mu_p = jnp.mean(P, axis=0, keepdims=True)
    Pc = P - mu_p
    nrm_p = jnp.sqrt(jnp.sum(Pc * Pc, axis=0, keepdims=True)) + 1e-12
    Yn = Pc / nrm_p

    X = st_ref[...]
    mu = jnp.mean(X, axis=0, keepdims=True)
    Xc = X - mu
    nrm = jnp.sqrt(jnp.sum(Xc * Xc, axis=0, keepdims=True)) + 1e-12
    Xn = Xc / nrm
    G = jax.lax.dot_general(Xn, Yn, (((0,), (0,)), ((), ())),
                            preferred_element_type=jnp.float32)
    B = gg_ref[...]
    na = jnp.sqrt(jnp.sum(G * G, axis=1, keepdims=True)) + 1e-12
    nb = jnp.sqrt(jnp.sum(B * B, axis=1, keepdims=True)) + 1e-12
    c = jnp.sum((G / na) * (B / nb), axis=1)
    out_ref[...] = jnp.sum(1.0 - c).reshape(1, 1)


def _genegraph(ST_fit, imp, gg):
    spec = lambda s: pl.BlockSpec(s, lambda: (0, 0))
    return pl.pallas_call(
        _gg_body,
        in_specs=[spec((NS, DI)), spec((NS, DS)), spec((DI, DS))],
        out_specs=spec((1, 1)),
        out_shape=_f32(1, 1),
        compiler_params=pltpu.CompilerParams(vmem_limit_bytes=110 << 20),
    )(ST_fit, imp, gg)


def kernel(ST_fit, ST_supervision, ST_edge, SC_fit, SC_supervision, SC_label,
           SC_genegraph, params):
    p = params
    h, r, es, ed, scl = _encoders(ST_fit, SC_fit, p)
    es = es[:, 0]
    ed = ed[:, 0]

    src, dst = ST_edge[0], ST_edge[1]
    logits = jax.nn.leaky_relu(es[src] + ed[dst], 0.2)
    m = jax.ops.segment_max(logits, dst, num_segments=NS)
    m = jnp.where(jnp.isfinite(m), m, 0.0)
    e = jnp.exp(logits - m[dst])
    s = jax.ops.segment_sum(e, dst, num_segments=NS)
    alpha = e / (s[dst] + 1e-16)
    agg = jax.ops.segment_sum(alpha[:, None] * h[src], dst, num_segments=NS)

    stl, mmd, clf = _latent_mmd_clf(agg, r, scl, SC_label, p)

    q, k = stl, scl
    d2 = (jnp.sum(q * q, 1)[:, None] + jnp.sum(k * k, 1)[None, :]
          - 2.0 * (q @ k.T))
    d2 = jnp.maximum(d2, 0.0)
    _, idx = jax.lax.top_k(-d2, KK)
    d_sel = jnp.take_along_axis(d2, idx, axis=1)
    w = jax.nn.softmax(-jnp.sqrt(d_sel + 1e-12), axis=1)
    rows = jnp.arange(NS)[:, None]
    Wmat = jnp.zeros((NS, NS), jnp.float32).at[rows, idx].set(w)
    imp = Wmat @ SC_supervision

    rec, cos, impl = _decoders(stl, scl, ST_fit, SC_fit, ST_supervision,
                               imp, p)
    gg = _genegraph(ST_fit, imp, SC_genegraph)

    loss_recon = rec[0, 0] / (NS * DI)
    loss_mmd = mmd[0, 0]
    loss_clf = clf[0, 0]
    loss_cos = cos[0, 0] / NS
    loss_impute = impl[0, 0] / NS
    loss_genegraph = gg[0, 0] / DI
    loss = (loss_recon + loss_mmd + loss_cos + loss_clf + loss_impute
            + loss_genegraph)
    return (loss, loss_recon, loss_mmd, loss_cos, loss_clf, loss_impute,
            loss_genegraph)


# speedup vs baseline: 1.9692x; 1.9365x over previous
import functools
import jax
import jax.numpy as jnp
from jax.experimental import pallas as pl
from jax.experimental.pallas import tpu as pltpu

DI = 2000
DH = 512
DL = 32
NC = 20
KK = 50
NS = 2048
NE = 12288
DS = 1000
RB = 256
NBLK = NS // RB


def _f32(*shape):
    return jax.ShapeDtypeStruct(shape, jnp.float32)


def _enc_body(st_ref, sc_ref, wg_ref, wr_ref, asrc_ref, adst_ref,
              we1_ref, be1_ref, we2_ref, be2_ref,
              h_ref, r_ref, es_ref, ed_ref, scl_ref):
    x = st_ref[...]
    h = jnp.dot(x, wg_ref[...], preferred_element_type=jnp.float32)
    h_ref[...] = h
    r_ref[...] = jnp.dot(x, wr_ref[...], preferred_element_type=jnp.float32)
    es_ref[...] = jnp.dot(h, asrc_ref[...], preferred_element_type=jnp.float32)
    ed_ref[...] = jnp.dot(h, adst_ref[...], preferred_element_type=jnp.float32)
    xc = sc_ref[...]
    hc = jnp.maximum(
        jnp.dot(xc, we1_ref[...], preferred_element_type=jnp.float32)
        + be1_ref[...], 0.0)
    scl_ref[...] = (jnp.dot(hc, we2_ref[...], preferred_element_type=jnp.float32)
                    + be2_ref[...])


def _encoders(ST_fit, SC_fit, p):
    blk = lambda i: (i, 0)
    full = lambda i: (0, 0)
    return pl.pallas_call(
        _enc_body,
        grid=(NBLK,),
        in_specs=[
            pl.BlockSpec((RB, DI), blk),
            pl.BlockSpec((RB, DI), blk),
            pl.BlockSpec((DI, DH), full),
            pl.BlockSpec((DI, DH), full),
            pl.BlockSpec((DH, 1), full),
            pl.BlockSpec((DH, 1), full),
            pl.BlockSpec((DI, DH), full),
            pl.BlockSpec((1, DH), full),
            pl.BlockSpec((DH, DL), full),
            pl.BlockSpec((1, DL), full),
        ],
        out_specs=[
            pl.BlockSpec((RB, DH), blk),
            pl.BlockSpec((RB, DH), blk),
            pl.BlockSpec((RB, 1), blk),
            pl.BlockSpec((RB, 1), blk),
            pl.BlockSpec((RB, DL), blk),
        ],
        out_shape=[_f32(NS, DH), _f32(NS, DH), _f32(NS, 1), _f32(NS, 1),
                   _f32(NS, DL)],
        compiler_params=pltpu.CompilerParams(vmem_limit_bytes=100 << 20),
    )(ST_fit, SC_fit, p['W_gat'], p['W_res'],
      p['a_src'].reshape(DH, 1), p['a_dst'].reshape(DH, 1),
      p['We1'], p['be1'].reshape(1, DH), p['We2'], p['be2'].reshape(1, DL))


def _lat_body(agg_ref, r_ref, wl_ref, bl_ref, scl_ref, lab_ref,
              wc1_ref, bc1_ref, wc2_ref, bc2_ref,
              stl_ref, mmd_ref, clf_ref):
    pre = agg_ref[...] + r_ref[...]
    hout = jnp.where(pre > 0, pre, jnp.exp(jnp.minimum(pre, 0.0)) - 1.0)
    stl = (jnp.dot(hout, wl_ref[...], preferred_element_type=jnp.float32)
           + bl_ref[...])
    stl_ref[...] = stl

    scl = scl_ref[...]
    t1 = jnp.maximum(
        jnp.dot(scl, wc1_ref[...], preferred_element_type=jnp.float32)
        + bc1_ref[...], 0.0)
    logits = (jnp.dot(t1, wc2_ref[...], preferred_element_type=jnp.float32)
              + bc2_ref[...])
    mx = jnp.max(logits, axis=1, keepdims=True)
    sh = logits - mx
    ls = sh - jnp.log(jnp.sum(jnp.exp(sh), axis=1, keepdims=True))
    cols = jax.lax.broadcasted_iota(jnp.int32, (NS, NC), 1)
    sel = jnp.where(cols == lab_ref[...], ls, 0.0)
    clf_ref[...] = (-jnp.sum(sel) / NS).reshape(1, 1)

    q = stl
    k = scl
    qs = jnp.sum(q * q, axis=1, keepdims=True)
    ks = jnp.sum(k * k, axis=1, keepdims=True)

    def dblock(a, b, sb):
        sa = jnp.sum(a * a, axis=1, keepdims=True)
        ab = jax.lax.dot_general(
            a, b, (((1,), (1,)), ((), ())), preferred_element_type=jnp.float32)
        return jnp.maximum(sa + sb.reshape(1, NS) - 2.0 * ab, 0.0)

    def bw_step(i, acc):
        qb = stl_ref[pl.ds(i * RB, RB), :]
        d = dblock(qb, k, ks)
        return acc + jnp.sum(d)

    bwsum = jax.lax.fori_loop(0, NBLK, bw_step, 0.0)
    bw = bwsum / (NS * NS) + 1e-12

    def exp_step(i, accs):
        axx, ayy, axy = accs
        qb = stl_ref[pl.ds(i * RB, RB), :]
        kb = scl_ref[pl.ds(i * RB, RB), :]
        dxx = dblock(qb, q, qs)
        dyy = dblock(kb, k, ks)
        dxy = dblock(qb, k, ks)
        axx = axx + jnp.sum(jnp.exp(-dxx / bw))
        ayy = ayy + jnp.sum(jnp.exp(-dyy / bw))
        axy = axy + jnp.sum(jnp.exp(-dxy / bw))
        return axx, ayy, axy

    sxx, syy, sxy = jax.lax.fori_loop(0, NBLK, exp_step, (0.0, 0.0, 0.0))
    mmd_ref[...] = ((sxx + syy - 2.0 * sxy) / (NS * NS)).reshape(1, 1)


def _latent_mmd_clf(agg, r, scl, lab, p):
    full = lambda: (0, 0)
    spec = lambda s: pl.BlockSpec(s, lambda: (0, 0))
    return pl.pallas_call(
        _lat_body,
        in_specs=[spec((NS, DH)), spec((NS, DH)), spec((DH, DL)),
                  spec((1, DL)), spec((NS, DL)), spec((NS, 1)),
                  spec((DL, DH)), spec((1, DH)), spec((DH, NC)),
                  spec((1, NC))],
        out_specs=[spec((NS, DL)), spec((1, 1)), spec((1, 1))],
        out_shape=[_f32(NS, DL), _f32(1, 1), _f32(1, 1)],
        compiler_params=pltpu.CompilerParams(vmem_limit_bytes=100 << 20),
    )(agg, r, p['W_lat'], p['b_lat'].reshape(1, DL), scl,
      lab.reshape(NS, 1), p['Wc1'], p['bc1'].reshape(1, DH),
      p['Wc2'], p['bc2'].reshape(1, NC))


def _dec_body(stl_ref, scl_ref, st_ref, sc_ref, sup_ref, imp_ref,
              w1st_ref, b1st_ref, w2st_ref, b2st_ref,
              w1sc_ref, b1sc_ref, w2sc_ref, b2sc_ref,
              rec_ref, cos_ref, imp_out_ref):
    i = pl.program_id(0)

    def dec(z, w1, b1, w2, b2):
        d1 = jnp.maximum(
            jnp.dot(z, w1, preferred_element_type=jnp.float32) + b1, 0.0)
        return jnp.dot(d1, w2, preferred_element_type=jnp.float32) + b2

    z = stl_ref[...]
    z2 = scl_ref[...]
    xst = st_ref[...]
    xsc = sc_ref[...]
    w1st = w1st_ref[...]; b1st = b1st_ref[...]
    w2st = w2st_ref[...]; b2st = b2st_ref[...]
    w1sc = w1sc_ref[...]; b1sc = b1sc_ref[...]
    w2sc = w2sc_ref[...]; b2sc = b2sc_ref[...]

    rec_st = dec(z, w1st, b1st, w2st, b2st)
    rec_sc = dec(z2, w1sc, b1sc, w2sc, b2sc)
    d1 = xst - rec_st
    d2 = xsc - rec_sc
    rsum = jnp.sum(d1 * d1) + jnp.sum(d2 * d2)

    def cosrows(a, b):
        na = jnp.sqrt(jnp.sum(a * a, axis=1, keepdims=True)) + 1e-12
        nb = jnp.sqrt(jnp.sum(b * b, axis=1, keepdims=True)) + 1e-12
        c = jnp.sum((a / na) * (b / nb), axis=1)
        return jnp.sum(1.0 - c)

    cross_sc = dec(z, w1sc, b1sc, w2sc, b2sc)
    cross_st = dec(z2, w1st, b1st, w2st, b2st)
    csum = cosrows(cross_sc, xst) + cosrows(cross_st, xsc)

    P = imp_ref[...]
    T = sup_ref[...]
    Pc = P - jnp.mean(P, axis=1, keepdims=True)
    Tc = T - jnp.mean(T, axis=1, keepdims=True)
    num = jnp.sum(Pc * Tc, axis=1)
    den = (jnp.sqrt(jnp.sum(Pc * Pc, axis=1)) *
           jnp.sqrt(jnp.sum(Tc * Tc, axis=1)) + 1e-12)
    isum = jnp.sum(1.0 - num / den)

    @pl.when(i == 0)
    def _():
        rec_ref[...] = jnp.zeros((1, 1), jnp.float32)
        cos_ref[...] = jnp.zeros((1, 1), jnp.float32)
        imp_out_ref[...] = jnp.zeros((1, 1), jnp.float32)

    rec_ref[...] += rsum.reshape(1, 1)
    cos_ref[...] += csum.reshape(1, 1)
    imp_out_ref[...] += isum.reshape(1, 1)


def _decoders(stl, scl, ST_fit, SC_fit, sup, imp, p):
    blk = lambda i: (i, 0)
    full = lambda i: (0, 0)
    return pl.pallas_call(
        _dec_body,
        grid=(NBLK,),
        in_specs=[
            pl.BlockSpec((RB, DL), blk),
            pl.BlockSpec((RB, DL), blk),
            pl.BlockSpec((RB, DI), blk),
            pl.BlockSpec((RB, DI), blk),
            pl.BlockSpec((RB, DS), blk),
            pl.BlockSpec((RB, DS), blk),
            pl.BlockSpec((DL, DH), full),
            pl.BlockSpec((1, DH), full),
            pl.BlockSpec((DH, DI), full),
            pl.BlockSpec((1, DI), full),
            pl.BlockSpec((DL, DH), full),
            pl.BlockSpec((1, DH), full),
            pl.BlockSpec((DH, DI), full),
            pl.BlockSpec((1, DI), full),
        ],
        out_specs=[pl.BlockSpec((1, 1), full)] * 3,
        out_shape=[_f32(1, 1)] * 3,
        compiler_params=pltpu.CompilerParams(vmem_limit_bytes=100 << 20),
    )(stl, scl, ST_fit, SC_fit, sup, imp,
      p['Wd1_st'], p['bd1_st'].reshape(1, DH), p['Wd2_st'],
      p['bd2_st'].reshape(1, DI),
      p['Wd1_sc'], p['bd1_sc'].reshape(1, DH), p['Wd2_sc'],
      p['bd2_sc'].reshape(1, DI))


def _gg_body(st_ref, imp_ref, gg_ref, out_ref):
    P = imp_ref[...]
    mu_p = jnp.mean(P, axis=0, keepdims=True)
    Pc = P - mu_p
    nrm_p = jnp.sqrt(jnp.sum(Pc * Pc, axis=0, keepdims=True)) + 1e-12
    Yn = Pc / nrm_p

    X = st_ref[...]
    mu = jnp.mean(X, axis=0, keepdims=True)
    Xc = X - mu
    nrm = jnp.sqrt(jnp.sum(Xc * Xc, axis=0, keepdims=True)) + 1e-12
    Xn = Xc / nrm
    G = jax.lax.dot_general(Xn, Yn, (((0,), (0,)), ((), ())),
                            preferred_element_type=jnp.float32)
    B = gg_ref[...]
    na = jnp.sqrt(jnp.sum(G * G, axis=1, keepdims=True)) + 1e-12
    nb = jnp.sqrt(jnp.sum(B * B, axis=1, keepdims=True)) + 1e-12
    c = jnp.sum((G / na) * (B / nb), axis=1)
    out_ref[...] = jnp.sum(1.0 - c).reshape(1, 1)


def _genegraph(ST_fit, imp, gg):
    spec = lambda s: pl.BlockSpec(s, lambda: (0, 0))
    return pl.pallas_call(
        _gg_body,
        in_specs=[spec((NS, DI)), spec((NS, DS)), spec((DI, DS))],
        out_specs=spec((1, 1)),
        out_shape=_f32(1, 1),
        compiler_params=pltpu.CompilerParams(vmem_limit_bytes=110 << 20),
    )(ST_fit, imp, gg)


def _attn_body(q_ref, k_ref, v_ref, imp_ref):
    qb = q_ref[...]
    k = k_ref[...]
    qs = jnp.sum(qb * qb, axis=1, keepdims=True)
    ks = jnp.sum(k * k, axis=1, keepdims=True)
    qk = jax.lax.dot_general(qb, k, (((1,), (1,)), ((), ())),
                             preferred_element_type=jnp.float32)
    D = jnp.maximum(qs + ks.reshape(1, NS) - 2.0 * qk, 0.0)

    u = jax.lax.bitcast_convert_type(D, jnp.int32)
    ones = jnp.ones((NS, 1), jnp.float32)

    def count(mask):
        return jax.lax.dot_general(
            jnp.where(mask, 1.0, 0.0), ones, (((1,), (0,)), ((), ())),
            preferred_element_type=jnp.float32)

    def vstep(i, p):
        cand = jnp.bitwise_or(p, jnp.left_shift(1, 30 - i))
        cnt = count(u < cand)
        return jnp.where(cnt <= KK - 1.0, cand, p)

    p = jax.lax.fori_loop(0, 31, vstep, jnp.zeros((RB, 1), jnp.int32))

    cnt_less = count(u < p)
    need = KK - cnt_less
    eq = u == p
    col = jax.lax.broadcasted_iota(jnp.int32, (RB, NS), 1)

    def cstep(i, c):
        cand = jnp.bitwise_or(c, jnp.left_shift(1, 10 - i))
        cnt = count(eq & (col < cand))
        return jnp.where(cnt <= need - 1.0, cand, c)

    c = jax.lax.fori_loop(0, 11, cstep, jnp.zeros((RB, 1), jnp.int32))
    sel = (u < p) | (eq & (col <= c))

    mn = jnp.min(D, axis=1, keepdims=True)
    x = jnp.sqrt(mn + 1e-12) - jnp.sqrt(D + 1e-12)
    e = jnp.where(sel, jnp.exp(x), 0.0)
    z = jax.lax.dot_general(e, ones, (((1,), (0,)), ((), ())),
                            preferred_element_type=jnp.float32)
    W = e / z
    imp_ref[...] = jnp.dot(W, v_ref[...], preferred_element_type=jnp.float32)


def _attn(stl, scl, V):
    blk = lambda i: (i, 0)
    full = lambda i: (0, 0)
    return pl.pallas_call(
        _attn_body,
        grid=(NBLK,),
        in_specs=[
            pl.BlockSpec((RB, DL), blk),
            pl.BlockSpec((NS, DL), full),
            pl.BlockSpec((NS, DS), full),
        ],
        out_specs=pl.BlockSpec((RB, DS), blk),
        out_shape=_f32(NS, DS),
        compiler_params=pltpu.CompilerParams(vmem_limit_bytes=100 << 20),
    )(stl, scl, V)


def kernel(ST_fit, ST_supervision, ST_edge, SC_fit, SC_supervision, SC_label,
           SC_genegraph, params):
    p = params
    h, r, es, ed, scl = _encoders(ST_fit, SC_fit, p)
    es = es[:, 0]
    ed = ed[:, 0]

    src, dst = ST_edge[0], ST_edge[1]
    logits = jax.nn.leaky_relu(es[src] + ed[dst], 0.2)
    m = jax.ops.segment_max(logits, dst, num_segments=NS)
    m = jnp.where(jnp.isfinite(m), m, 0.0)
    e = jnp.exp(logits - m[dst])
    s = jax.ops.segment_sum(e, dst, num_segments=NS)
    alpha = e / (s[dst] + 1e-16)
    agg = jax.ops.segment_sum(alpha[:, None] * h[src], dst, num_segments=NS)

    stl, mmd, clf = _latent_mmd_clf(agg, r, scl, SC_label, p)

    imp = _attn(stl, scl, SC_supervision)

    rec, cos, impl = _decoders(stl, scl, ST_fit, SC_fit, ST_supervision,
                               imp, p)
    gg = _genegraph(ST_fit, imp, SC_genegraph)

    loss_recon = rec[0, 0] / (NS * DI)
    loss_mmd = mmd[0, 0]
    loss_clf = clf[0, 0]
    loss_cos = cos[0, 0] / NS
    loss_impute = impl[0, 0] / NS
    loss_genegraph = gg[0, 0] / DI
    loss = (loss_recon + loss_mmd + loss_cos + loss_clf + loss_impute
            + loss_genegraph)
    return (loss, loss_recon, loss_mmd, loss_cos, loss_clf, loss_impute,
            loss_genegraph)


# speedup vs baseline: 4.1293x; 2.0969x over previous
import functools
import jax
import jax.numpy as jnp
from jax import lax
from jax.experimental import pallas as pl
from jax.experimental.pallas import tpu as pltpu
from jax.experimental.pallas import tpu_sc as plsc

DI = 2000
DH = 512
DL = 32
NC = 20
KK = 50
NS = 2048
NE = 12288
DS = 1000
RB = 256
NBLK = NS // RB


def _f32(*shape):
    return jax.ShapeDtypeStruct(shape, jnp.float32)


def _enc_body(st_ref, sc_ref, wg_ref, wr_ref, asrc_ref, adst_ref,
              we1_ref, be1_ref, we2_ref, be2_ref,
              h_ref, r_ref, es_ref, ed_ref, scl_ref):
    x = st_ref[...]
    h = jnp.dot(x, wg_ref[...], preferred_element_type=jnp.float32)
    h_ref[...] = h
    r_ref[...] = jnp.dot(x, wr_ref[...], preferred_element_type=jnp.float32)
    es_ref[...] = jnp.dot(h, asrc_ref[...], preferred_element_type=jnp.float32)
    ed_ref[...] = jnp.dot(h, adst_ref[...], preferred_element_type=jnp.float32)
    xc = sc_ref[...]
    hc = jnp.maximum(
        jnp.dot(xc, we1_ref[...], preferred_element_type=jnp.float32)
        + be1_ref[...], 0.0)
    scl_ref[...] = (jnp.dot(hc, we2_ref[...], preferred_element_type=jnp.float32)
                    + be2_ref[...])


def _encoders(ST_fit, SC_fit, p):
    blk = lambda i: (i, 0)
    full = lambda i: (0, 0)
    return pl.pallas_call(
        _enc_body,
        grid=(NBLK,),
        in_specs=[
            pl.BlockSpec((RB, DI), blk),
            pl.BlockSpec((RB, DI), blk),
            pl.BlockSpec((DI, DH), full),
            pl.BlockSpec((DI, DH), full),
            pl.BlockSpec((DH, 1), full),
            pl.BlockSpec((DH, 1), full),
            pl.BlockSpec((DI, DH), full),
            pl.BlockSpec((1, DH), full),
            pl.BlockSpec((DH, DL), full),
            pl.BlockSpec((1, DL), full),
        ],
        out_specs=[
            pl.BlockSpec((RB, DH), blk),
            pl.BlockSpec((RB, DH), blk),
            pl.BlockSpec((RB, 1), blk),
            pl.BlockSpec((RB, 1), blk),
            pl.BlockSpec((RB, DL), blk),
        ],
        out_shape=[_f32(NS, DH), _f32(NS, DH), _f32(NS, 1), _f32(NS, 1),
                   _f32(NS, DL)],
        compiler_params=pltpu.CompilerParams(vmem_limit_bytes=100 << 20),
    )(ST_fit, SC_fit, p['W_gat'], p['W_res'],
      p['a_src'].reshape(DH, 1), p['a_dst'].reshape(DH, 1),
      p['We1'], p['be1'].reshape(1, DH), p['We2'], p['be2'].reshape(1, DL))


EPT_A = NE // 16
EPT_B = NE // 32


def _gat_body(src_hbm, dst_hbm, es_hbm, ed_hbm, h_hbm,
              zs_hbm, out_hbm,
              srcA, dstA, es_v, ed_v, s_part, s_full, srcB, dstB, alpha_v,
              rows, s_grid_sh):
    c = lax.axis_index("c")
    t = lax.axis_index("s")
    w = c * 16 + t
    iota = lax.iota(jnp.int32, 16)
    zi16 = jnp.zeros((16,), jnp.int32)

    pltpu.sync_copy(src_hbm.at[pl.ds(t * EPT_A, EPT_A)], srcA)
    pltpu.sync_copy(dst_hbm.at[pl.ds(t * EPT_A, EPT_A)], dstA)
    pltpu.sync_copy(es_hbm, es_v)
    pltpu.sync_copy(ed_hbm, ed_v)
    pltpu.sync_copy(src_hbm.at[pl.ds(w * EPT_B, EPT_B)], srcB)
    pltpu.sync_copy(dst_hbm.at[pl.ds(w * EPT_B, EPT_B)], dstB)
    pltpu.sync_copy(zs_hbm, s_part)

    def edge_e(s16, d16):
        g = plsc.load_gather(es_v, [s16]) + plsc.load_gather(ed_v, [d16])
        l = jnp.where(g >= 0.0, g, 0.2 * g)
        return jnp.exp(l)

    def pa(i, carry):
        s16 = srcA[pl.ds(i * 16, 16)]
        d16 = dstA[pl.ds(i * 16, 16)]
        e16 = edge_e(s16, d16)
        cur = plsc.load_gather(s_part, [iota, d16])
        plsc.store_scatter(s_part, [iota, d16], cur + e16)
        return carry

    lax.fori_loop(0, EPT_A // 16, pa, 0)

    def reduce16(chn, carry):
        acc = s_part[0, pl.ds(chn * 16, 16)]
        for r in range(1, 16):
            acc = acc + s_part[r, pl.ds(chn * 16, 16)]
        s_full[0, pl.ds(chn * 16, 16)] = acc
        return carry

    lax.fori_loop(0, NS // 16, reduce16, 0)
    pltpu.sync_copy(s_full, s_grid_sh.at[pl.ds(t, 1)])
    plsc.subcore_barrier()
    pltpu.sync_copy(s_grid_sh, s_part)
    lax.fori_loop(0, NS // 16, reduce16, 0)

    def pal(i, carry):
        s16 = srcB[pl.ds(i * 16, 16)]
        d16 = dstB[pl.ds(i * 16, 16)]
        e16 = edge_e(s16, d16)
        sv = plsc.load_gather(s_full, [zi16, d16])
        alpha_v[pl.ds(i * 16, 16)] = e16 / (sv + 1e-16)
        return carry

    lax.fori_loop(0, EPT_B // 16, pal, 0)

    def pb(ch, carry):
        for k in range(4):
            s16 = srcB[pl.ds(ch * 64 + k * 16, 16)]
            pltpu.sync_copy(h_hbm.at[s16], rows.at[pl.ds(k * 16, 16)])

        def scale(e, carry2):
            av = plsc.load_gather(
                alpha_v, [jnp.full((16,), ch * 64 + e, jnp.int32)])
            for j in range(DH // 16):
                rows[e, pl.ds(j * 16, 16)] = rows[e, pl.ds(j * 16, 16)] * av
            return carry2

        lax.fori_loop(0, 64, scale, 0)
        pltpu.sync_copy(rows, out_hbm.at[pl.ds(w * EPT_B + ch * 64, 64)])
        return carry

    lax.fori_loop(0, EPT_B // 64, pb, 0)


def _gat_sc(src, dst, es, ed, h):
    zs = jnp.zeros((16, NS), jnp.float32)
    mesh = plsc.VectorSubcoreMesh(core_axis_name="c", subcore_axis_name="s")
    f = functools.partial(
        pl.kernel,
        mesh=mesh,
        out_type=jax.ShapeDtypeStruct((NE, DH), jnp.float32),
        compiler_params=pltpu.CompilerParams(needs_layout_passes=False),
        scratch_types=[
            pltpu.VMEM((EPT_A,), jnp.int32),
            pltpu.VMEM((EPT_A,), jnp.int32),
            pltpu.VMEM((NS,), jnp.float32),
            pltpu.VMEM((NS,), jnp.float32),
            pltpu.VMEM((16, NS), jnp.float32),
            pltpu.VMEM((1, NS), jnp.float32),
            pltpu.VMEM((EPT_B,), jnp.int32),
            pltpu.VMEM((EPT_B,), jnp.int32),
            pltpu.VMEM((EPT_B,), jnp.float32),
            pltpu.VMEM((64, DH), jnp.float32),
            pltpu.VMEM_SHARED((16, NS), jnp.float32),
        ],
    )(_gat_body)
    return f(src, dst, es, ed, h, zs)


ECH = NS


def _agg_body(g_ref, dst_ref, agg_ref):
    i = pl.program_id(0)
    dstc = dst_ref[...].reshape(1, ECH)
    rowi = jax.lax.broadcasted_iota(jnp.int32, (NS, ECH), 0)
    oh = jnp.where(rowi == dstc, 1.0, 0.0)
    part = jnp.dot(oh, g_ref[...], preferred_element_type=jnp.float32)

    @pl.when(i == 0)
    def _():
        agg_ref[...] = jnp.zeros((NS, DH), jnp.float32)

    agg_ref[...] += part


def _agg(G, dst):
    return pl.pallas_call(
        _agg_body,
        grid=(NE // ECH,),
        in_specs=[
            pl.BlockSpec((ECH, DH), lambda i: (i, 0)),
            pl.BlockSpec((1, 1, ECH), lambda i: (i, 0, 0)),
        ],
        out_specs=pl.BlockSpec((NS, DH), lambda i: (0, 0)),
        out_shape=_f32(NS, DH),
        compiler_params=pltpu.CompilerParams(vmem_limit_bytes=100 << 20),
    )(G, dst.reshape(NE // ECH, 1, ECH))


def _lat_body(agg_ref, r_ref, wl_ref, bl_ref, scl_ref, lab_ref,
              wc1_ref, bc1_ref, wc2_ref, bc2_ref,
              stl_ref, mmd_ref, clf_ref):
    pre = agg_ref[...] + r_ref[...]
    hout = jnp.where(pre > 0, pre, jnp.exp(jnp.minimum(pre, 0.0)) - 1.0)
    stl = (jnp.dot(hout, wl_ref[...], preferred_element_type=jnp.float32)
           + bl_ref[...])
    stl_ref[...] = stl

    scl = scl_ref[...]
    t1 = jnp.maximum(
        jnp.dot(scl, wc1_ref[...], preferred_element_type=jnp.float32)
        + bc1_ref[...], 0.0)
    logits = (jnp.dot(t1, wc2_ref[...], preferred_element_type=jnp.float32)
              + bc2_ref[...])
    mx = jnp.max(logits, axis=1, keepdims=True)
    sh = logits - mx
    ls = sh - jnp.log(jnp.sum(jnp.exp(sh), axis=1, keepdims=True))
    cols = jax.lax.broadcasted_iota(jnp.int32, (NS, NC), 1)
    sel = jnp.where(cols == lab_ref[...], ls, 0.0)
    clf_ref[...] = (-jnp.sum(sel) / NS).reshape(1, 1)

    q = stl
    k = scl
    qs = jnp.sum(q * q, axis=1, keepdims=True)
    ks = jnp.sum(k * k, axis=1, keepdims=True)

    def dblock(a, b, sb):
        sa = jnp.sum(a * a, axis=1, keepdims=True)
        ab = jax.lax.dot_general(
            a, b, (((1,), (1,)), ((), ())), preferred_element_type=jnp.float32)
        return jnp.maximum(sa + sb.reshape(1, NS) - 2.0 * ab, 0.0)

    def bw_step(i, acc):
        qb = stl_ref[pl.ds(i * RB, RB), :]
        d = dblock(qb, k, ks)
        return acc + jnp.sum(d)

    bwsum = jax.lax.fori_loop(0, NBLK, bw_step, 0.0)
    bw = bwsum / (NS * NS) + 1e-12

    def exp_step(i, accs):
        axx, ayy, axy = accs
        qb = stl_ref[pl.ds(i * RB, RB), :]
        kb = scl_ref[pl.ds(i * RB, RB), :]
        dxx = dblock(qb, q, qs)
        dyy = dblock(kb, k, ks)
        dxy = dblock(qb, k, ks)
        axx = axx + jnp.sum(jnp.exp(-dxx / bw))
        ayy = ayy + jnp.sum(jnp.exp(-dyy / bw))
        axy = axy + jnp.sum(jnp.exp(-dxy / bw))
        return axx, ayy, axy

    sxx, syy, sxy = jax.lax.fori_loop(0, NBLK, exp_step, (0.0, 0.0, 0.0))
    mmd_ref[...] = ((sxx + syy - 2.0 * sxy) / (NS * NS)).reshape(1, 1)


def _latent_mmd_clf(agg, r, scl, lab, p):
    full = lambda: (0, 0)
    spec = lambda s: pl.BlockSpec(s, lambda: (0, 0))
    return pl.pallas_call(
        _lat_body,
        in_specs=[spec((NS, DH)), spec((NS, DH)), spec((DH, DL)),
                  spec((1, DL)), spec((NS, DL)), spec((NS, 1)),
                  spec((DL, DH)), spec((1, DH)), spec((DH, NC)),
                  spec((1, NC))],
        out_specs=[spec((NS, DL)), spec((1, 1)), spec((1, 1))],
        out_shape=[_f32(NS, DL), _f32(1, 1), _f32(1, 1)],
        compiler_params=pltpu.CompilerParams(vmem_limit_bytes=100 << 20),
    )(agg, r, p['W_lat'], p['b_lat'].reshape(1, DL), scl,
      lab.reshape(NS, 1), p['Wc1'], p['bc1'].reshape(1, DH),
      p['Wc2'], p['bc2'].reshape(1, NC))


def _dec_body(stl_ref, scl_ref, st_ref, sc_ref, sup_ref, imp_ref,
              w1st_ref, b1st_ref, w2st_ref, b2st_ref,
              w1sc_ref, b1sc_ref, w2sc_ref, b2sc_ref,
              rec_ref, cos_ref, imp_out_ref):
    i = pl.program_id(0)

    def dec(z, w1, b1, w2, b2):
        d1 = jnp.maximum(
            jnp.dot(z, w1, preferred_element_type=jnp.float32) + b1, 0.0)
        return jnp.dot(d1, w2, preferred_element_type=jnp.float32) + b2

    z = stl_ref[...]
    z2 = scl_ref[...]
    xst = st_ref[...]
    xsc = sc_ref[...]
    w1st = w1st_ref[...]; b1st = b1st_ref[...]
    w2st = w2st_ref[...]; b2st = b2st_ref[...]
    w1sc = w1sc_ref[...]; b1sc = b1sc_ref[...]
    w2sc = w2sc_ref[...]; b2sc = b2sc_ref[...]

    rec_st = dec(z, w1st, b1st, w2st, b2st)
    rec_sc = dec(z2, w1sc, b1sc, w2sc, b2sc)
    d1 = xst - rec_st
    d2 = xsc - rec_sc
    rsum = jnp.sum(d1 * d1) + jnp.sum(d2 * d2)

    def cosrows(a, b):
        na = jnp.sqrt(jnp.sum(a * a, axis=1, keepdims=True)) + 1e-12
        nb = jnp.sqrt(jnp.sum(b * b, axis=1, keepdims=True)) + 1e-12
        c = jnp.sum((a / na) * (b / nb), axis=1)
        return jnp.sum(1.0 - c)

    cross_sc = dec(z, w1sc, b1sc, w2sc, b2sc)
    cross_st = dec(z2, w1st, b1st, w2st, b2st)
    csum = cosrows(cross_sc, xst) + cosrows(cross_st, xsc)

    P = imp_ref[...]
    T = sup_ref[...]
    Pc = P - jnp.mean(P, axis=1, keepdims=True)
    Tc = T - jnp.mean(T, axis=1, keepdims=True)
    num = jnp.sum(Pc * Tc, axis=1)
    den = (jnp.sqrt(jnp.sum(Pc * Pc, axis=1)) *
           jnp.sqrt(jnp.sum(Tc * Tc, axis=1)) + 1e-12)
    isum = jnp.sum(1.0 - num / den)

    @pl.when(i == 0)
    def _():
        rec_ref[...] = jnp.zeros((1, 1), jnp.float32)
        cos_ref[...] = jnp.zeros((1, 1), jnp.float32)
        imp_out_ref[...] = jnp.zeros((1, 1), jnp.float32)

    rec_ref[...] += rsum.reshape(1, 1)
    cos_ref[...] += csum.reshape(1, 1)
    imp_out_ref[...] += isum.reshape(1, 1)


def _decoders(stl, scl, ST_fit, SC_fit, sup, imp, p):
    blk = lambda i: (i, 0)
    full = lambda i: (0, 0)
    return pl.pallas_call(
        _dec_body,
        grid=(NBLK,),
        in_specs=[
            pl.BlockSpec((RB, DL), blk),
            pl.BlockSpec((RB, DL), blk),
            pl.BlockSpec((RB, DI), blk),
            pl.BlockSpec((RB, DI), blk),
            pl.BlockSpec((RB, DS), blk),
            pl.BlockSpec((RB, DS), blk),
            pl.BlockSpec((DL, DH), full),
            pl.BlockSpec((1, DH), full),
            pl.BlockSpec((DH, DI), full),
            pl.BlockSpec((1, DI), full),
            pl.BlockSpec((DL, DH), full),
            pl.BlockSpec((1, DH), full),
            pl.BlockSpec((DH, DI), full),
            pl.BlockSpec((1, DI), full),
        ],
        out_specs=[pl.BlockSpec((1, 1), full)] * 3,
        out_shape=[_f32(1, 1)] * 3,
        compiler_params=pltpu.CompilerParams(vmem_limit_bytes=100 << 20),
    )(stl, scl, ST_fit, SC_fit, sup, imp,
      p['Wd1_st'], p['bd1_st'].reshape(1, DH), p['Wd2_st'],
      p['bd2_st'].reshape(1, DI),
      p['Wd1_sc'], p['bd1_sc'].reshape(1, DH), p['Wd2_sc'],
      p['bd2_sc'].reshape(1, DI))


def _gg_body(st_ref, imp_ref, gg_ref, out_ref):
    P = imp_ref[...]
    mu_p = jnp.mean(P, axis=0, keepdims=True)
    Pc = P - mu_p
    nrm_p = jnp.sqrt(jnp.sum(Pc * Pc, axis=0, keepdims=True)) + 1e-12
    Yn = Pc / nrm_p

    X = st_ref[...]
    mu = jnp.mean(X, axis=0, keepdims=True)
    Xc = X - mu
    nrm = jnp.sqrt(jnp.sum(Xc * Xc, axis=0, keepdims=True)) + 1e-12
    Xn = Xc / nrm
    G = jax.lax.dot_general(Xn, Yn, (((0,), (0,)), ((), ())),
                            preferred_element_type=jnp.float32)
    B = gg_ref[...]
    na = jnp.sqrt(jnp.sum(G * G, axis=1, keepdims=True)) + 1e-12
    nb = jnp.sqrt(jnp.sum(B * B, axis=1, keepdims=True)) + 1e-12
    c = jnp.sum((G / na) * (B / nb), axis=1)
    out_ref[...] = jnp.sum(1.0 - c).reshape(1, 1)


def _genegraph(ST_fit, imp, gg):
    spec = lambda s: pl.BlockSpec(s, lambda: (0, 0))
    return pl.pallas_call(
        _gg_body,
        in_specs=[spec((NS, DI)), spec((NS, DS)), spec((DI, DS))],
        out_specs=spec((1, 1)),
        out_shape=_f32(1, 1),
        compiler_params=pltpu.CompilerParams(vmem_limit_bytes=110 << 20),
    )(ST_fit, imp, gg)


def _attn_body(q_ref, k_ref, v_ref, imp_ref):
    qb = q_ref[...]
    k = k_ref[...]
    qs = jnp.sum(qb * qb, axis=1, keepdims=True)
    ks = jnp.sum(k * k, axis=1, keepdims=True)
    qk = jax.lax.dot_general(qb, k, (((1,), (1,)), ((), ())),
                             preferred_element_type=jnp.float32)
    D = jnp.maximum(qs + ks.reshape(1, NS) - 2.0 * qk, 0.0)

    u = jax.lax.bitcast_convert_type(D, jnp.int32)
    ones = jnp.ones((NS, 1), jnp.float32)

    def count(mask):
        return jax.lax.dot_general(
            jnp.where(mask, 1.0, 0.0), ones, (((1,), (0,)), ((), ())),
            preferred_element_type=jnp.float32)

    def vstep(i, p):
        cand = jnp.bitwise_or(p, jnp.left_shift(1, 30 - i))
        cnt = count(u < cand)
        return jnp.where(cnt <= KK - 1.0, cand, p)

    p = jax.lax.fori_loop(0, 31, vstep, jnp.zeros((RB, 1), jnp.int32))

    cnt_less = count(u < p)
    need = KK - cnt_less
    eq = u == p
    col = jax.lax.broadcasted_iota(jnp.int32, (RB, NS), 1)

    def cstep(i, c):
        cand = jnp.bitwise_or(c, jnp.left_shift(1, 10 - i))
        cnt = count(eq & (col < cand))
        return jnp.where(cnt <= need - 1.0, cand, c)

    c = jax.lax.fori_loop(0, 11, cstep, jnp.zeros((RB, 1), jnp.int32))
    sel = (u < p) | (eq & (col <= c))

    mn = jnp.min(D, axis=1, keepdims=True)
    x = jnp.sqrt(mn + 1e-12) - jnp.sqrt(D + 1e-12)
    e = jnp.where(sel, jnp.exp(x), 0.0)
    z = jax.lax.dot_general(e, ones, (((1,), (0,)), ((), ())),
                            preferred_element_type=jnp.float32)
    W = e / z
    imp_ref[...] = jnp.dot(W, v_ref[...], preferred_element_type=jnp.float32)


def _attn(stl, scl, V):
    blk = lambda i: (i, 0)
    full = lambda i: (0, 0)
    return pl.pallas_call(
        _attn_body,
        grid=(NBLK,),
        in_specs=[
            pl.BlockSpec((RB, DL), blk),
            pl.BlockSpec((NS, DL), full),
            pl.BlockSpec((NS, DS), full),
        ],
        out_specs=pl.BlockSpec((RB, DS), blk),
        out_shape=_f32(NS, DS),
        compiler_params=pltpu.CompilerParams(vmem_limit_bytes=100 << 20),
    )(stl, scl, V)


def kernel(ST_fit, ST_supervision, ST_edge, SC_fit, SC_supervision, SC_label,
           SC_genegraph, params):
    p = params
    h, r, es, ed, scl = _encoders(ST_fit, SC_fit, p)

    src = ST_edge[0].astype(jnp.int32)
    dst = ST_edge[1].astype(jnp.int32)
    G = _gat_sc(src, dst, es.reshape(NS), ed.reshape(NS), h)
    agg = _agg(G, dst)

    stl, mmd, clf = _latent_mmd_clf(agg, r, scl, SC_label, p)

    imp = _attn(stl, scl, SC_supervision)

    rec, cos, impl = _decoders(stl, scl, ST_fit, SC_fit, ST_supervision,
                               imp, p)
    gg = _genegraph(ST_fit, imp, SC_genegraph)

    loss_recon = rec[0, 0] / (NS * DI)
    loss_mmd = mmd[0, 0]
    loss_clf = clf[0, 0]
    loss_cos = cos[0, 0] / NS
    loss_impute = impl[0, 0] / NS
    loss_genegraph = gg[0, 0] / DI
    loss = (loss_recon + loss_mmd + loss_cos + loss_clf + loss_impute
            + loss_genegraph)
    return (loss, loss_recon, loss_mmd, loss_cos, loss_clf, loss_impute,
            loss_genegraph)
